# Initial kernel scaffold; baseline (speedup 1.0000x reference)
import jax
import jax.numpy as jnp
from jax.experimental import pallas as pl


def kernel(predictions, targets):
    raise NotImplementedError("write your pallas kernel here")


# speedup vs baseline: 1.3341x; 1.3341x over previous
import functools

import numpy as np
import jax
import jax.numpy as jnp
from jax import lax
from jax.experimental import pallas as pl
from jax.experimental.pallas import tpu as pltpu
from jax.experimental.pallas import tpu_sc as plsc

_MARGIN = 0.1
_LANES = 16


def _pair_indices(batch_size: int):
    n_pairs = min(1000, batch_size * (batch_size - 1) // 2)
    ki, kj = jax.random.split(jax.random.key(42))
    idx_i = jax.random.randint(ki, (n_pairs,), 0, batch_size).astype(jnp.int32)
    idx_j = jax.random.randint(kj, (n_pairs,), 0, batch_size).astype(jnp.int32)
    n_pad = -(-n_pairs // _LANES) * _LANES
    pad = n_pad - n_pairs
    if pad:
        zeros = jnp.zeros((pad,), jnp.int32)
        idx_i = jnp.concatenate([idx_i, zeros])
        idx_j = jnp.concatenate([idx_j, zeros])
    return idx_i, idx_j


@functools.lru_cache(maxsize=None)
def _build_sc_kernel(batch_size: int, n_pad: int):
    n_chunks = n_pad // _LANES
    mesh = plsc.VectorSubcoreMesh(core_axis_name="c", subcore_axis_name="s")

    @functools.partial(
        pl.kernel,
        out_type=jax.ShapeDtypeStruct((_LANES,), jnp.float32),
        mesh=mesh,
        compiler_params=pltpu.CompilerParams(needs_layout_passes=False),
        scratch_types=[
            pltpu.VMEM((batch_size,), jnp.float32),
            pltpu.VMEM((batch_size,), jnp.float32),
            pltpu.VMEM((n_pad,), jnp.int32),
            pltpu.VMEM((n_pad,), jnp.int32),
            pltpu.VMEM((_LANES,), jnp.float32),
        ],
    )
    def sc_loss(pred_hbm, targ_hbm, ii_hbm, jj_hbm, out_hbm,
                pred_v, targ_v, ii_v, jj_v, out_v):
        cid = lax.axis_index("c")
        sid = lax.axis_index("s")

        @pl.when(jnp.logical_and(cid == 0, sid == 0))
        def _():
            pltpu.sync_copy(pred_hbm, pred_v)
            pltpu.sync_copy(targ_hbm, targ_v)
            pltpu.sync_copy(ii_hbm, ii_v)
            pltpu.sync_copy(jj_hbm, jj_v)

            def step(k, carry):
                acc, cnt = carry
                ii = ii_v[pl.ds(k * _LANES, _LANES)]
                jj = jj_v[pl.ds(k * _LANES, _LANES)]
                ti = plsc.load_gather(targ_v, [ii])
                tj = plsc.load_gather(targ_v, [jj])
                pi = plsc.load_gather(pred_v, [ii])
                pj = plsc.load_gather(pred_v, [jj])
                y = jnp.sign(ti - tj)
                m = jnp.where((ii != jj) & (ti != tj), 1.0, 0.0)
                per = jnp.maximum(0.0, -y * (pi - pj) + _MARGIN)
                return acc + per * m, cnt + m

            acc, cnt = lax.fori_loop(
                0, n_chunks, step,
                (jnp.zeros((_LANES,), jnp.float32),
                 jnp.zeros((_LANES,), jnp.float32)))
            total = jnp.full((_LANES,), jnp.sum(acc), dtype=jnp.float32)
            denom = jnp.full((_LANES,), jnp.sum(cnt), dtype=jnp.float32)
            out_v[...] = total / jnp.maximum(denom, 1.0)
            pltpu.sync_copy(out_v, out_hbm)

    return sc_loss


def kernel(predictions, targets):
    batch_size = predictions.shape[0]
    if batch_size < 2:
        return jnp.asarray(0.0, dtype=jnp.float32)
    ii, jj = _pair_indices(batch_size)
    sc_loss = _build_sc_kernel(batch_size, ii.shape[0])
    out = sc_loss(predictions, targets, ii, jj)
    return out[0]


# speedup vs baseline: 1.3409x; 1.0051x over previous
import functools

import jax
import jax.numpy as jnp
from jax import lax
from jax.experimental import pallas as pl
from jax.experimental.pallas import tpu as pltpu
from jax.experimental.pallas import tpu_sc as plsc

_MARGIN = 0.1
_LANES = 16
_N_SUBCORES = 16


def _pair_indices(batch_size: int, n_pad: int):
    n_pairs = min(1000, batch_size * (batch_size - 1) // 2)
    ki, kj = jax.random.split(jax.random.key(42))
    idx_i = jax.random.randint(ki, (n_pairs,), 0, batch_size).astype(jnp.int32)
    idx_j = jax.random.randint(kj, (n_pairs,), 0, batch_size).astype(jnp.int32)
    zeros = jnp.zeros((n_pad - n_pairs,), jnp.int32)
    return jnp.concatenate([idx_i, zeros]), jnp.concatenate([idx_j, zeros])


@functools.lru_cache(maxsize=None)
def _build_sc_kernel(batch_size: int, n_pad: int):
    per_w = n_pad // _N_SUBCORES
    n_chunks = per_w // _LANES
    mesh = plsc.VectorSubcoreMesh(
        core_axis_name="c", subcore_axis_name="s", num_cores=1)

    @functools.partial(
        pl.kernel,
        out_type=jax.ShapeDtypeStruct((_LANES,), jnp.float32),
        mesh=mesh,
        compiler_params=pltpu.CompilerParams(needs_layout_passes=False),
        scratch_types=[
            pltpu.VMEM((batch_size,), jnp.float32),
            pltpu.VMEM((batch_size,), jnp.float32),
            pltpu.VMEM((per_w,), jnp.int32),
            pltpu.VMEM((per_w,), jnp.int32),
            pltpu.VMEM((2 * _LANES,), jnp.float32),
            pltpu.VMEM_SHARED((_N_SUBCORES * 2 * _LANES,), jnp.float32),
            pltpu.VMEM((_N_SUBCORES * 2 * _LANES,), jnp.float32),
            pltpu.VMEM((_LANES,), jnp.float32),
        ],
    )
    def sc_loss(pred_hbm, targ_hbm, ii_hbm, jj_hbm, out_hbm,
                pred_v, targ_v, ii_v, jj_v,
                partial_v, shared, all_v, out_v):
        sid = lax.axis_index("s")
        base = sid * per_w

        pltpu.sync_copy(ii_hbm.at[pl.ds(base, per_w)], ii_v)
        pltpu.sync_copy(jj_hbm.at[pl.ds(base, per_w)], jj_v)
        pltpu.sync_copy(pred_hbm, pred_v)
        pltpu.sync_copy(targ_hbm, targ_v)

        acc = jnp.zeros((_LANES,), jnp.float32)
        cnt = jnp.zeros((_LANES,), jnp.float32)
        for k in range(n_chunks):
            sl = pl.ds(k * _LANES, _LANES)
            ii, jj = ii_v[sl], jj_v[sl]
            ti = plsc.load_gather(targ_v, [ii])
            tj = plsc.load_gather(targ_v, [jj])
            pi = plsc.load_gather(pred_v, [ii])
            pj = plsc.load_gather(pred_v, [jj])
            y = jnp.sign(ti - tj)
            m = jnp.where((ii != jj) & (ti != tj), 1.0, 0.0)
            per = jnp.maximum(0.0, -y * (pi - pj) + _MARGIN)
            acc = acc + per * m
            cnt = cnt + m

        partial_v[pl.ds(0, _LANES)] = acc
        partial_v[pl.ds(_LANES, _LANES)] = cnt
        pltpu.sync_copy(partial_v, shared.at[pl.ds(sid * 2 * _LANES, 2 * _LANES)])
        plsc.subcore_barrier()

        @pl.when(sid == 0)
        def _():
            pltpu.sync_copy(shared, all_v)
            acc = jnp.zeros((_LANES,), jnp.float32)
            cnt = jnp.zeros((_LANES,), jnp.float32)
            for w in range(_N_SUBCORES):
                acc = acc + all_v[pl.ds(w * 2 * _LANES, _LANES)]
                cnt = cnt + all_v[pl.ds(w * 2 * _LANES + _LANES, _LANES)]
            total = jnp.full((_LANES,), jnp.sum(acc), dtype=jnp.float32)
            denom = jnp.full((_LANES,), jnp.sum(cnt), dtype=jnp.float32)
            out_v[...] = total / jnp.maximum(denom, 1.0)
            pltpu.sync_copy(out_v, out_hbm)

    return sc_loss


def kernel(predictions, targets):
    batch_size = predictions.shape[0]
    if batch_size < 2:
        return jnp.asarray(0.0, dtype=jnp.float32)
    n_pairs = min(1000, batch_size * (batch_size - 1) // 2)
    n_pad = -(-n_pairs // (_N_SUBCORES * _LANES)) * (_N_SUBCORES * _LANES)
    ii, jj = _pair_indices(batch_size, n_pad)
    sc_loss = _build_sc_kernel(batch_size, n_pad)
    out = sc_loss(predictions, targets, ii, jj)
    return out[0]


# speedup vs baseline: 1.4211x; 1.0598x over previous
import functools

import jax
import jax.numpy as jnp
from jax import lax
from jax.experimental import pallas as pl
from jax.experimental.pallas import tpu as pltpu
from jax.experimental.pallas import tpu_sc as plsc

_MARGIN = 0.1
_LANES = 16


def _pair_indices(batch_size: int, n_pad: int):
    n_pairs = min(1000, batch_size * (batch_size - 1) // 2)
    ki, kj = jax.random.split(jax.random.key(42))
    idx_i = jax.random.randint(ki, (n_pairs,), 0, batch_size).astype(jnp.int32)
    idx_j = jax.random.randint(kj, (n_pairs,), 0, batch_size).astype(jnp.int32)
    zeros = jnp.zeros((n_pad - n_pairs,), jnp.int32)
    return jnp.concatenate([idx_i, zeros]), jnp.concatenate([idx_j, zeros])


@functools.lru_cache(maxsize=None)
def _build_sc_kernel(batch_size: int, n_pad: int):
    n_chunks = n_pad // _LANES
    mesh = plsc.VectorSubcoreMesh(
        core_axis_name="c", subcore_axis_name="s", num_cores=1)

    @functools.partial(
        pl.kernel,
        out_type=jax.ShapeDtypeStruct((1,), jnp.float32),
        mesh=mesh,
        compiler_params=pltpu.CompilerParams(needs_layout_passes=False),
        scratch_types=[
            pltpu.VMEM((batch_size,), jnp.float32),
            pltpu.VMEM((batch_size,), jnp.float32),
            pltpu.VMEM((n_pad,), jnp.int32),
            pltpu.VMEM((n_pad,), jnp.int32),
            pltpu.VMEM((_LANES,), jnp.float32),
            pltpu.SemaphoreType.DMA,
            pltpu.SemaphoreType.DMA,
            pltpu.SemaphoreType.DMA,
            pltpu.SemaphoreType.DMA,
        ],
    )
    def sc_loss(pred_hbm, targ_hbm, ii_hbm, jj_hbm, out_hbm,
                pred_v, targ_v, ii_v, jj_v, out_v,
                sem0, sem1, sem2, sem3):
        sid = lax.axis_index("s")

        @pl.when(sid == 0)
        def _():
            cps = [
                pltpu.async_copy(ii_hbm, ii_v, sem0),
                pltpu.async_copy(jj_hbm, jj_v, sem1),
                pltpu.async_copy(pred_hbm, pred_v, sem2),
                pltpu.async_copy(targ_hbm, targ_v, sem3),
            ]
            for cp in cps:
                cp.wait()

            acc = jnp.zeros((_LANES,), jnp.float32)
            cnt = jnp.zeros((_LANES,), jnp.float32)
            for k in range(n_chunks):
                sl = pl.ds(k * _LANES, _LANES)
                ii, jj = ii_v[sl], jj_v[sl]
                ti = plsc.load_gather(targ_v, [ii])
                tj = plsc.load_gather(targ_v, [jj])
                pi = plsc.load_gather(pred_v, [ii])
                pj = plsc.load_gather(pred_v, [jj])
                y = jnp.sign(ti - tj)
                m = jnp.where((ii != jj) & (ti != tj), 1.0, 0.0)
                per = jnp.maximum(0.0, -y * (pi - pj) + _MARGIN)
                acc = acc + per * m
                cnt = cnt + m

            total = jnp.full((_LANES,), jnp.sum(acc), dtype=jnp.float32)
            denom = jnp.full((_LANES,), jnp.sum(cnt), dtype=jnp.float32)
            out_v[...] = total / jnp.maximum(denom, 1.0)
            pltpu.sync_copy(out_v.at[pl.ds(0, 1)], out_hbm)

    return sc_loss


def kernel(predictions, targets):
    batch_size = predictions.shape[0]
    if batch_size < 2:
        return jnp.asarray(0.0, dtype=jnp.float32)
    n_pairs = min(1000, batch_size * (batch_size - 1) // 2)
    n_pad = -(-n_pairs // _LANES) * _LANES
    ii, jj = _pair_indices(batch_size, n_pad)
    sc_loss = _build_sc_kernel(batch_size, n_pad)
    out = sc_loss(predictions, targets, ii, jj)
    return jnp.reshape(out, ())
